# Initial kernel scaffold; baseline (speedup 1.0000x reference)
import jax
import jax.numpy as jnp
from jax.experimental import pallas as pl


def kernel(x, edge_index, W1, att_src1, att_dst1, b1, W2, att_src2, att_dst2, b2):
    raise NotImplementedError("write your pallas kernel here")


# speedup vs baseline: 45.3742x; 45.3742x over previous
import functools

import jax
import jax.numpy as jnp
from jax import lax
from jax.experimental import pallas as pl
from jax.experimental.pallas import tpu as pltpu
from jax.experimental.pallas import tpu_sc as plsc

F32 = jnp.float32

N_NODES = 10000
N_EDGES = 320000
F_IN = 128
HEADS = 8
CH1 = 8
HC = HEADS * CH1
NCLS = 40

SC_CORES = 2
SC_SUBCORES = 16
SC_WORKERS = SC_CORES * SC_SUBCORES
EDGE_BLOCK = 80

TC_BLOCK = 1000


def _tc1_body(x_ref, w_ref, ast_ref, adt_ref, t1_ref, a1_ref, init_ref):
    h = jnp.dot(x_ref[...], w_ref[...], preferred_element_type=F32)
    a_s = jnp.dot(h, ast_ref[...], preferred_element_type=F32)
    a_d = jnp.dot(h, adt_ref[...], preferred_element_type=F32)
    t1_ref[...] = jnp.concatenate([h, a_s, a_s], axis=1)
    a1_ref[...] = jnp.concatenate([a_d, a_d], axis=1)
    z = a_s + a_d
    w = jnp.exp(jnp.maximum(z, 0.2 * z))
    w8 = jnp.concatenate([w] * 8, axis=1)
    init_ref[...] = jnp.concatenate([h * w8, w, w], axis=1)


def _tc1(x, w1t, ast, adt):
    nb = N_NODES // TC_BLOCK
    return pl.pallas_call(
        _tc1_body,
        grid=(nb,),
        in_specs=[
            pl.BlockSpec((TC_BLOCK, F_IN), lambda i: (i, 0)),
            pl.BlockSpec((F_IN, HC), lambda i: (0, 0)),
            pl.BlockSpec((HC, HEADS), lambda i: (0, 0)),
            pl.BlockSpec((HC, HEADS), lambda i: (0, 0)),
        ],
        out_specs=[
            pl.BlockSpec((TC_BLOCK, 80), lambda i: (i, 0)),
            pl.BlockSpec((TC_BLOCK, 16), lambda i: (i, 0)),
            pl.BlockSpec((TC_BLOCK, 80), lambda i: (i, 0)),
        ],
        out_shape=[
            jax.ShapeDtypeStruct((N_NODES, 80), F32),
            jax.ShapeDtypeStruct((N_NODES, 16), F32),
            jax.ShapeDtypeStruct((N_NODES, 80), F32),
        ],
    )(x, w1t, ast, adt)


def _tc2_body(p_ref, init_ref, b1_ref, w2_ref, att2_ref,
              t2_ref, a2_ref, init2_ref):
    acc = p_ref[0] + p_ref[1] + init_ref[...]
    denom = acc[:, 64:72] + 1e-16
    dtile = jnp.concatenate([denom] * 8, axis=1)
    out1 = acc[:, :64] / dtile + b1_ref[...]
    h2 = jnp.where(out1 > 0, out1, jnp.exp(out1) - 1.0)
    h2r = jnp.dot(h2, w2_ref[...], preferred_element_type=F32)
    a2 = jnp.dot(h2r, att2_ref[...], preferred_element_type=F32)
    a2s = a2[:, 0:1]
    a2d = a2[:, 1:2]
    zeros8 = jnp.zeros((h2r.shape[0], 8), F32)
    t2_ref[...] = jnp.concatenate([h2r, zeros8] + [a2s] * 16, axis=1)
    a2_ref[...] = jnp.concatenate([a2d] * 16, axis=1)
    z = a2s + a2d
    w = jnp.exp(jnp.maximum(z, 0.2 * z))
    init2_ref[...] = jnp.concatenate([h2r * w, zeros8] + [w] * 16, axis=1)


def _tc2(p, init1, b1t, w2t, att2m):
    nb = N_NODES // TC_BLOCK
    return pl.pallas_call(
        _tc2_body,
        grid=(nb,),
        in_specs=[
            pl.BlockSpec((SC_CORES, TC_BLOCK, 80), lambda i: (0, i, 0)),
            pl.BlockSpec((TC_BLOCK, 80), lambda i: (i, 0)),
            pl.BlockSpec((1, HC), lambda i: (0, 0)),
            pl.BlockSpec((HC, NCLS), lambda i: (0, 0)),
            pl.BlockSpec((NCLS, 2), lambda i: (0, 0)),
        ],
        out_specs=[
            pl.BlockSpec((TC_BLOCK, 64), lambda i: (i, 0)),
            pl.BlockSpec((TC_BLOCK, 16), lambda i: (i, 0)),
            pl.BlockSpec((TC_BLOCK, 64), lambda i: (i, 0)),
        ],
        out_shape=[
            jax.ShapeDtypeStruct((N_NODES, 64), F32),
            jax.ShapeDtypeStruct((N_NODES, 16), F32),
            jax.ShapeDtypeStruct((N_NODES, 64), F32),
        ],
    )(p, init1, b1t, w2t, att2m)


def _tc3_body(q_ref, init2_ref, b2_ref, out_ref):
    acc = q_ref[0] + q_ref[1] + init2_ref[...]
    denom = acc[:, 48:49] + 1e-16
    logits = acc[:, :40] / denom + b2_ref[...]
    m = jnp.max(logits, axis=1, keepdims=True)
    lse = jnp.log(jnp.sum(jnp.exp(logits - m), axis=1, keepdims=True)) + m
    out_ref[...] = logits - lse


def _tc3(q, init2, b2r):
    nb = N_NODES // TC_BLOCK
    return pl.pallas_call(
        _tc3_body,
        grid=(nb,),
        in_specs=[
            pl.BlockSpec((SC_CORES, TC_BLOCK, 64), lambda i: (0, i, 0)),
            pl.BlockSpec((TC_BLOCK, 64), lambda i: (i, 0)),
            pl.BlockSpec((1, NCLS), lambda i: (0, 0)),
        ],
        out_specs=pl.BlockSpec((TC_BLOCK, NCLS), lambda i: (i, 0)),
        out_shape=jax.ShapeDtypeStruct((N_NODES, NCLS), F32),
    )(q, init2, b2r)


def _make_sc_edge_pass(width, eoff):
    per_w = N_EDGES // SC_WORKERS
    nblk = per_w // EDGE_BLOCK
    nfeat = eoff // 16
    rps = 624
    last_off = rps * (SC_SUBCORES - 1)
    last_n = N_NODES - last_off

    mesh = plsc.VectorSubcoreMesh(core_axis_name="c", subcore_axis_name="s")

    @functools.partial(
        pl.kernel,
        mesh=mesh,
        compiler_params=pltpu.CompilerParams(use_tc_tiling_on_sc=False),
        out_type=jax.ShapeDtypeStruct((SC_CORES, N_NODES, width), F32),
        scratch_types=[
            pltpu.VMEM((EDGE_BLOCK,), jnp.int32),
            pltpu.VMEM((EDGE_BLOCK,), jnp.int32),
            pltpu.VMEM((EDGE_BLOCK, width), F32),
            pltpu.VMEM((EDGE_BLOCK, 16), F32),
            pltpu.VMEM((EDGE_BLOCK, width), F32),
            pltpu.VMEM_SHARED((N_NODES, width), F32),
        ],
    )
    def sc_edge_pass(t_hbm, a_hbm, src_hbm, dst_hbm, zero_hbm, p_hbm,
                     sidx, didx, rows, arows, wbuf, acc):
        cid = lax.axis_index("c")
        sid = lax.axis_index("s")
        wid = sid * SC_CORES + cid

        @pl.when(sid < SC_SUBCORES - 1)
        def _zero_main():
            pltpu.sync_copy(zero_hbm.at[pl.ds(sid * rps, rps)],
                            acc.at[pl.ds(sid * rps, rps)])

        @pl.when(sid == SC_SUBCORES - 1)
        def _zero_last():
            pltpu.sync_copy(zero_hbm.at[pl.ds(last_off, last_n)],
                            acc.at[pl.ds(last_off, last_n)])

        plsc.subcore_barrier()

        @pl.loop(0, nblk)
        def _blk(j):
            base = wid * per_w + j * EDGE_BLOCK
            pltpu.sync_copy(src_hbm.at[pl.ds(base, EDGE_BLOCK)], sidx)
            pltpu.sync_copy(dst_hbm.at[pl.ds(base, EDGE_BLOCK)], didx)
            pltpu.sync_copy(t_hbm.at[sidx], rows)
            pltpu.sync_copy(a_hbm.at[didx], arows)

            @pl.loop(0, EDGE_BLOCK)
            def _edge(i):
                z = rows[i, pl.ds(eoff, 16)] + arows[i, pl.ds(0, 16)]
                w = jnp.exp(jnp.maximum(z, 0.2 * z))
                wbuf[i, pl.ds(eoff, 16)] = w
                for k in range(nfeat):
                    wbuf[i, pl.ds(16 * k, 16)] = rows[i, pl.ds(16 * k, 16)] * w

            pltpu.sync_copy(wbuf, acc.at[didx], add=True)

        plsc.subcore_barrier()

        @pl.when(sid < SC_SUBCORES - 1)
        def _drain_main():
            pltpu.sync_copy(acc.at[pl.ds(sid * rps, rps)],
                            p_hbm.at[cid, pl.ds(sid * rps, rps)])

        @pl.when(sid == SC_SUBCORES - 1)
        def _drain_last():
            pltpu.sync_copy(acc.at[pl.ds(last_off, last_n)],
                            p_hbm.at[cid, pl.ds(last_off, last_n)])

    return sc_edge_pass


_sc_cache = {}


def _sc_pass(width, eoff, *args):
    key = (width, eoff)
    if key not in _sc_cache:
        _sc_cache[key] = _make_sc_edge_pass(width, eoff)
    return _sc_cache[key](*args)


def kernel(x, edge_index, W1, att_src1, att_dst1, b1,
           W2, att_src2, att_dst2, b2):
    perm = jnp.arange(HC)
    perm = (perm % 8) * 8 + perm // 8
    w1t = W1[:, perm]
    eye8 = jnp.eye(8, dtype=F32)
    ast = (att_src1[0].T[:, :, None] * eye8[None, :, :]).reshape(HC, HEADS)
    adt = (att_dst1[0].T[:, :, None] * eye8[None, :, :]).reshape(HC, HEADS)
    b1t = b1[perm].reshape(1, HC)
    w2t = W2[perm, :]
    att2m = jnp.concatenate(
        [att_src2[0, 0][:, None], att_dst2[0, 0][:, None]], axis=1)
    b2r = b2.reshape(1, NCLS)
    src = edge_index[0]
    dst = edge_index[1]
    zeros80 = jnp.zeros((N_NODES, 80), F32)
    zeros64 = jnp.zeros((N_NODES, 64), F32)

    t1, a1, init1 = _tc1(x, w1t, ast, adt)
    p = _sc_pass(80, 64, t1, a1, src, dst, zeros80)
    t2, a2, init2 = _tc2(p, init1, b1t, w2t, att2m)
    q = _sc_pass(64, 48, t2, a2, src, dst, zeros64)
    return _tc3(q, init2, b2r)


# speedup vs baseline: 99.2031x; 2.1863x over previous
import functools

import jax
import jax.numpy as jnp
from jax import lax
from jax.experimental import pallas as pl
from jax.experimental.pallas import tpu as pltpu
from jax.experimental.pallas import tpu_sc as plsc

F32 = jnp.float32

N_NODES = 10000
N_EDGES = 320000
F_IN = 128
HEADS = 8
CH1 = 8
HC = HEADS * CH1
NCLS = 40

SC_CORES = 2
SC_SUBCORES = 16
SC_WORKERS = SC_CORES * SC_SUBCORES
EDGE_BLOCK = 80

TC_BLOCK = 1000


def _tc1_body(x_ref, w_ref, ast_ref, adt_ref, t1_ref, a1_ref, init_ref):
    h = jnp.dot(x_ref[...], w_ref[...], preferred_element_type=F32)
    a_s = jnp.dot(h, ast_ref[...], preferred_element_type=F32)
    a_d = jnp.dot(h, adt_ref[...], preferred_element_type=F32)
    t1_ref[...] = jnp.concatenate([h, a_s, a_s], axis=1)
    a1_ref[...] = jnp.concatenate([a_d, a_d], axis=1)
    z = a_s + a_d
    w = jnp.exp(jnp.maximum(z, 0.2 * z))
    w8 = jnp.concatenate([w] * 8, axis=1)
    init_ref[...] = jnp.concatenate([h * w8, w, w], axis=1)


def _tc1(x, w1t, ast, adt):
    nb = N_NODES // TC_BLOCK
    return pl.pallas_call(
        _tc1_body,
        grid=(nb,),
        in_specs=[
            pl.BlockSpec((TC_BLOCK, F_IN), lambda i: (i, 0)),
            pl.BlockSpec((F_IN, HC), lambda i: (0, 0)),
            pl.BlockSpec((HC, HEADS), lambda i: (0, 0)),
            pl.BlockSpec((HC, HEADS), lambda i: (0, 0)),
        ],
        out_specs=[
            pl.BlockSpec((TC_BLOCK, 80), lambda i: (i, 0)),
            pl.BlockSpec((TC_BLOCK, 16), lambda i: (i, 0)),
            pl.BlockSpec((TC_BLOCK, 80), lambda i: (i, 0)),
        ],
        out_shape=[
            jax.ShapeDtypeStruct((N_NODES, 80), F32),
            jax.ShapeDtypeStruct((N_NODES, 16), F32),
            jax.ShapeDtypeStruct((N_NODES, 80), F32),
        ],
    )(x, w1t, ast, adt)


def _tc2_body(p_ref, init_ref, b1_ref, w2_ref, att2_ref,
              t2_ref, a2_ref, init2_ref):
    acc = p_ref[0] + p_ref[1] + init_ref[...]
    denom = acc[:, 64:72] + 1e-16
    dtile = jnp.concatenate([denom] * 8, axis=1)
    out1 = acc[:, :64] / dtile + b1_ref[...]
    h2 = jnp.where(out1 > 0, out1, jnp.exp(out1) - 1.0)
    h2r = jnp.dot(h2, w2_ref[...], preferred_element_type=F32)
    a2 = jnp.dot(h2r, att2_ref[...], preferred_element_type=F32)
    a2s = a2[:, 0:1]
    a2d = a2[:, 1:2]
    zeros8 = jnp.zeros((h2r.shape[0], 8), F32)
    t2_ref[...] = jnp.concatenate([h2r, zeros8] + [a2s] * 16, axis=1)
    a2_ref[...] = jnp.concatenate([a2d] * 16, axis=1)
    z = a2s + a2d
    w = jnp.exp(jnp.maximum(z, 0.2 * z))
    init2_ref[...] = jnp.concatenate([h2r * w, zeros8] + [w] * 16, axis=1)


def _tc2(p, init1, b1t, w2t, att2m):
    nb = N_NODES // TC_BLOCK
    return pl.pallas_call(
        _tc2_body,
        grid=(nb,),
        in_specs=[
            pl.BlockSpec((SC_CORES, TC_BLOCK, 80), lambda i: (0, i, 0)),
            pl.BlockSpec((TC_BLOCK, 80), lambda i: (i, 0)),
            pl.BlockSpec((1, HC), lambda i: (0, 0)),
            pl.BlockSpec((HC, NCLS), lambda i: (0, 0)),
            pl.BlockSpec((NCLS, 2), lambda i: (0, 0)),
        ],
        out_specs=[
            pl.BlockSpec((TC_BLOCK, 64), lambda i: (i, 0)),
            pl.BlockSpec((TC_BLOCK, 16), lambda i: (i, 0)),
            pl.BlockSpec((TC_BLOCK, 64), lambda i: (i, 0)),
        ],
        out_shape=[
            jax.ShapeDtypeStruct((N_NODES, 64), F32),
            jax.ShapeDtypeStruct((N_NODES, 16), F32),
            jax.ShapeDtypeStruct((N_NODES, 64), F32),
        ],
    )(p, init1, b1t, w2t, att2m)


def _tc3_body(q_ref, init2_ref, b2_ref, out_ref):
    acc = q_ref[0] + q_ref[1] + init2_ref[...]
    denom = acc[:, 48:49] + 1e-16
    logits = acc[:, :40] / denom + b2_ref[...]
    m = jnp.max(logits, axis=1, keepdims=True)
    lse = jnp.log(jnp.sum(jnp.exp(logits - m), axis=1, keepdims=True)) + m
    out_ref[...] = logits - lse


def _tc3(q, init2, b2r):
    nb = N_NODES // TC_BLOCK
    return pl.pallas_call(
        _tc3_body,
        grid=(nb,),
        in_specs=[
            pl.BlockSpec((SC_CORES, TC_BLOCK, 64), lambda i: (0, i, 0)),
            pl.BlockSpec((TC_BLOCK, 64), lambda i: (i, 0)),
            pl.BlockSpec((1, NCLS), lambda i: (0, 0)),
        ],
        out_specs=pl.BlockSpec((TC_BLOCK, NCLS), lambda i: (i, 0)),
        out_shape=jax.ShapeDtypeStruct((N_NODES, NCLS), F32),
    )(q, init2, b2r)


def _make_sc_edge_pass(width, eoff):
    per_w = N_EDGES // SC_WORKERS
    nblk = per_w // EDGE_BLOCK
    nfeat = eoff // 16
    rps = 624
    last_off = rps * (SC_SUBCORES - 1)
    last_n = N_NODES - last_off

    mesh = plsc.VectorSubcoreMesh(core_axis_name="c", subcore_axis_name="s")

    B = EDGE_BLOCK
    npairs = nblk // 2

    @functools.partial(
        pl.kernel,
        mesh=mesh,
        compiler_params=pltpu.CompilerParams(use_tc_tiling_on_sc=False),
        out_type=jax.ShapeDtypeStruct((SC_CORES, N_NODES, width), F32),
        scratch_types=[
            pltpu.VMEM((B,), jnp.int32), pltpu.VMEM((B,), jnp.int32),
            pltpu.VMEM((B,), jnp.int32), pltpu.VMEM((B,), jnp.int32),
            pltpu.VMEM((B,), jnp.int32), pltpu.VMEM((B,), jnp.int32),
            pltpu.VMEM((B, width), F32), pltpu.VMEM((B, width), F32),
            pltpu.VMEM((B, 16), F32), pltpu.VMEM((B, 16), F32),
            pltpu.VMEM((B, width), F32), pltpu.VMEM((B, width), F32),
            pltpu.VMEM_SHARED((N_NODES, width), F32),
            pltpu.SemaphoreType.DMA, pltpu.SemaphoreType.DMA,
            pltpu.SemaphoreType.DMA, pltpu.SemaphoreType.DMA,
            pltpu.SemaphoreType.DMA, pltpu.SemaphoreType.DMA,
        ],
    )
    def sc_edge_pass(t_hbm, a_hbm, src_hbm, dst_hbm, zero_hbm, p_hbm,
                     sidx0, sidx1, didx0, didx1, sdidx0, sdidx1,
                     rows0, rows1, arows0, arows1, wbuf0, wbuf1, acc,
                     semi0, semi1, semg0, semg1, sems0, sems1):
        cid = lax.axis_index("c")
        sid = lax.axis_index("s")
        wid = sid * SC_CORES + cid
        ebase = wid * per_w

        sidx = (sidx0, sidx1)
        didx = (didx0, didx1)
        sdidx = (sdidx0, sdidx1)
        rows = (rows0, rows1)
        arows = (arows0, arows1)
        wbuf = (wbuf0, wbuf1)
        semi = (semi0, semi1)
        semg = (semg0, semg1)
        sems = (sems0, sems1)

        def issue_idx(j, b):
            base = ebase + j * B
            pltpu.async_copy(src_hbm.at[pl.ds(base, B)], sidx[b], semi[b])
            pltpu.async_copy(dst_hbm.at[pl.ds(base, B)], didx[b], semi[b])

        def wait_idx(b):
            pltpu.make_async_copy(src_hbm.at[pl.ds(0, B)], sidx[b],
                                  semi[b]).wait()
            pltpu.make_async_copy(dst_hbm.at[pl.ds(0, B)], didx[b],
                                  semi[b]).wait()

        def issue_gather(b):
            pltpu.async_copy(t_hbm.at[sidx[b]], rows[b], semg[b])
            pltpu.async_copy(a_hbm.at[didx[b]], arows[b], semg[b])

        def wait_gather(b):
            pltpu.make_async_copy(t_hbm.at[sidx[b]], rows[b], semg[b]).wait()
            pltpu.make_async_copy(a_hbm.at[didx[b]], arows[b], semg[b]).wait()

        def copy_didx(b):
            for k in range(B // 16):
                sdidx[b][pl.ds(16 * k, 16)] = didx[b][pl.ds(16 * k, 16)]

        def compute(b):
            rb, ab, wb = rows[b], arows[b], wbuf[b]

            @pl.loop(0, B)
            def _edge(i):
                z = rb[i, pl.ds(eoff, 16)] + ab[i, pl.ds(0, 16)]
                w = jnp.exp(jnp.maximum(z, 0.2 * z))
                wb[i, pl.ds(eoff, 16)] = w
                for k in range(nfeat):
                    wb[i, pl.ds(16 * k, 16)] = rb[i, pl.ds(16 * k, 16)] * w

        def issue_scatter(b):
            pltpu.async_copy(wbuf[b], acc.at[sdidx[b]], sems[b], add=True)

        def wait_scatter(b):
            pltpu.make_async_copy(wbuf[b], acc.at[sdidx[b]], sems[b]).wait()

        @pl.when(sid < SC_SUBCORES - 1)
        def _zero_main():
            pltpu.sync_copy(zero_hbm.at[pl.ds(sid * rps, rps)],
                            acc.at[pl.ds(sid * rps, rps)])

        @pl.when(sid == SC_SUBCORES - 1)
        def _zero_last():
            pltpu.sync_copy(zero_hbm.at[pl.ds(last_off, last_n)],
                            acc.at[pl.ds(last_off, last_n)])

        plsc.subcore_barrier()

        issue_idx(0, 0)
        issue_idx(1, 1)
        wait_idx(0)
        issue_gather(0)

        wait_gather(0)
        wait_idx(1)
        issue_gather(1)
        copy_didx(0)
        issue_idx(2, 0)
        compute(0)
        issue_scatter(0)

        wait_gather(1)
        wait_idx(0)
        issue_gather(0)
        copy_didx(1)
        issue_idx(3, 1)
        compute(1)
        issue_scatter(1)

        @pl.loop(1, npairs)
        def _pair(m):
            j0 = 2 * m
            wait_gather(0)
            wait_idx(1)
            issue_gather(1)
            wait_scatter(0)
            copy_didx(0)
            issue_idx(j0 + 2, 0)
            compute(0)
            issue_scatter(0)

            wait_gather(1)
            wait_idx(0)
            issue_gather(0)
            wait_scatter(1)
            copy_didx(1)

            @pl.when(m < npairs - 1)
            def _prefetch():
                issue_idx(j0 + 3, 1)

            compute(1)
            issue_scatter(1)

        wait_gather(0)
        wait_scatter(0)
        copy_didx(0)
        compute(0)
        issue_scatter(0)

        wait_scatter(1)
        wait_scatter(0)

        plsc.subcore_barrier()

        @pl.when(sid < SC_SUBCORES - 1)
        def _drain_main():
            pltpu.sync_copy(acc.at[pl.ds(sid * rps, rps)],
                            p_hbm.at[cid, pl.ds(sid * rps, rps)])

        @pl.when(sid == SC_SUBCORES - 1)
        def _drain_last():
            pltpu.sync_copy(acc.at[pl.ds(last_off, last_n)],
                            p_hbm.at[cid, pl.ds(last_off, last_n)])

    return sc_edge_pass


_sc_cache = {}


def _sc_pass(width, eoff, *args):
    key = (width, eoff)
    if key not in _sc_cache:
        _sc_cache[key] = _make_sc_edge_pass(width, eoff)
    return _sc_cache[key](*args)


def kernel(x, edge_index, W1, att_src1, att_dst1, b1,
           W2, att_src2, att_dst2, b2):
    perm = jnp.arange(HC)
    perm = (perm % 8) * 8 + perm // 8
    w1t = W1[:, perm]
    eye8 = jnp.eye(8, dtype=F32)
    ast = (att_src1[0].T[:, :, None] * eye8[None, :, :]).reshape(HC, HEADS)
    adt = (att_dst1[0].T[:, :, None] * eye8[None, :, :]).reshape(HC, HEADS)
    b1t = b1[perm].reshape(1, HC)
    w2t = W2[perm, :]
    att2m = jnp.concatenate(
        [att_src2[0, 0][:, None], att_dst2[0, 0][:, None]], axis=1)
    b2r = b2.reshape(1, NCLS)
    src = edge_index[0]
    dst = edge_index[1]
    zeros80 = jnp.zeros((N_NODES, 80), F32)
    zeros64 = jnp.zeros((N_NODES, 64), F32)

    t1, a1, init1 = _tc1(x, w1t, ast, adt)
    p = _sc_pass(80, 64, t1, a1, src, dst, zeros80)
    t2, a2, init2 = _tc2(p, init1, b1t, w2t, att2m)
    q = _sc_pass(64, 48, t2, a2, src, dst, zeros64)
    return _tc3(q, init2, b2r)


# speedup vs baseline: 114.7436x; 1.1567x over previous
import functools

import jax
import jax.numpy as jnp
from jax import lax
from jax.experimental import pallas as pl
from jax.experimental.pallas import tpu as pltpu
from jax.experimental.pallas import tpu_sc as plsc

F32 = jnp.float32

N_NODES = 10000
N_EDGES = 320000
F_IN = 128
HEADS = 8
CH1 = 8
HC = HEADS * CH1
NCLS = 40

SC_CORES = 2
SC_SUBCORES = 16
SC_WORKERS = SC_CORES * SC_SUBCORES
EDGE_BLOCK = 80

TC_BLOCK = 1000


def _tc1_body(x_ref, w_ref, ast_ref, adt_ref, t1_ref, a1_ref, init_ref):
    h = jnp.dot(x_ref[...], w_ref[...], preferred_element_type=F32)
    a_s = jnp.dot(h, ast_ref[...], preferred_element_type=F32)
    a_d = jnp.dot(h, adt_ref[...], preferred_element_type=F32)
    t1_ref[...] = jnp.concatenate([h, a_s, a_s], axis=1)
    a1_ref[...] = jnp.concatenate([a_d, a_d], axis=1)
    z = a_s + a_d
    w = jnp.exp(jnp.maximum(z, 0.2 * z))
    w8 = jnp.concatenate([w] * 8, axis=1)
    init_ref[...] = jnp.concatenate([h * w8, w, w], axis=1)


def _tc1(x, w1t, ast, adt):
    nb = N_NODES // TC_BLOCK
    return pl.pallas_call(
        _tc1_body,
        grid=(nb,),
        in_specs=[
            pl.BlockSpec((TC_BLOCK, F_IN), lambda i: (i, 0)),
            pl.BlockSpec((F_IN, HC), lambda i: (0, 0)),
            pl.BlockSpec((HC, HEADS), lambda i: (0, 0)),
            pl.BlockSpec((HC, HEADS), lambda i: (0, 0)),
        ],
        out_specs=[
            pl.BlockSpec((TC_BLOCK, 80), lambda i: (i, 0)),
            pl.BlockSpec((TC_BLOCK, 16), lambda i: (i, 0)),
            pl.BlockSpec((TC_BLOCK, 80), lambda i: (i, 0)),
        ],
        out_shape=[
            jax.ShapeDtypeStruct((N_NODES, 80), F32),
            jax.ShapeDtypeStruct((N_NODES, 16), F32),
            jax.ShapeDtypeStruct((N_NODES, 80), F32),
        ],
    )(x, w1t, ast, adt)


def _tc2_body(p_ref, init_ref, b1_ref, w2_ref, att2_ref,
              t2_ref, a2_ref, init2_ref):
    acc = p_ref[0] + p_ref[1] + init_ref[...]
    denom = acc[:, 64:72] + 1e-16
    dtile = jnp.concatenate([denom] * 8, axis=1)
    out1 = acc[:, :64] / dtile + b1_ref[...]
    h2 = jnp.where(out1 > 0, out1, jnp.exp(out1) - 1.0)
    h2r = jnp.dot(h2, w2_ref[...], preferred_element_type=F32)
    a2 = jnp.dot(h2r, att2_ref[...], preferred_element_type=F32)
    a2s = a2[:, 0:1]
    a2d = a2[:, 1:2]
    zeros8 = jnp.zeros((h2r.shape[0], 8), F32)
    t2_ref[...] = jnp.concatenate([h2r, zeros8] + [a2s] * 16, axis=1)
    a2_ref[...] = jnp.concatenate([a2d] * 16, axis=1)
    z = a2s + a2d
    w = jnp.exp(jnp.maximum(z, 0.2 * z))
    init2_ref[...] = jnp.concatenate([h2r * w, zeros8] + [w] * 16, axis=1)


def _tc2(p, init1, b1t, w2t, att2m):
    nb = N_NODES // TC_BLOCK
    return pl.pallas_call(
        _tc2_body,
        grid=(nb,),
        in_specs=[
            pl.BlockSpec((SC_CORES, TC_BLOCK, 80), lambda i: (0, i, 0)),
            pl.BlockSpec((TC_BLOCK, 80), lambda i: (i, 0)),
            pl.BlockSpec((1, HC), lambda i: (0, 0)),
            pl.BlockSpec((HC, NCLS), lambda i: (0, 0)),
            pl.BlockSpec((NCLS, 2), lambda i: (0, 0)),
        ],
        out_specs=[
            pl.BlockSpec((TC_BLOCK, 64), lambda i: (i, 0)),
            pl.BlockSpec((TC_BLOCK, 16), lambda i: (i, 0)),
            pl.BlockSpec((TC_BLOCK, 64), lambda i: (i, 0)),
        ],
        out_shape=[
            jax.ShapeDtypeStruct((N_NODES, 64), F32),
            jax.ShapeDtypeStruct((N_NODES, 16), F32),
            jax.ShapeDtypeStruct((N_NODES, 64), F32),
        ],
    )(p, init1, b1t, w2t, att2m)


def _tc3_body(q_ref, init2_ref, b2_ref, out_ref):
    acc = q_ref[0] + q_ref[1] + init2_ref[...]
    denom = acc[:, 48:49] + 1e-16
    logits = acc[:, :40] / denom + b2_ref[...]
    m = jnp.max(logits, axis=1, keepdims=True)
    lse = jnp.log(jnp.sum(jnp.exp(logits - m), axis=1, keepdims=True)) + m
    out_ref[...] = logits - lse


def _tc3(q, init2, b2r):
    nb = N_NODES // TC_BLOCK
    return pl.pallas_call(
        _tc3_body,
        grid=(nb,),
        in_specs=[
            pl.BlockSpec((SC_CORES, TC_BLOCK, 64), lambda i: (0, i, 0)),
            pl.BlockSpec((TC_BLOCK, 64), lambda i: (i, 0)),
            pl.BlockSpec((1, NCLS), lambda i: (0, 0)),
        ],
        out_specs=pl.BlockSpec((TC_BLOCK, NCLS), lambda i: (i, 0)),
        out_shape=jax.ShapeDtypeStruct((N_NODES, NCLS), F32),
    )(q, init2, b2r)


def _make_sc_edge_pass(width, eoff):
    per_w = N_EDGES // SC_WORKERS
    nblk = per_w // EDGE_BLOCK
    nfeat = eoff // 16
    rps = 624
    last_off = rps * (SC_SUBCORES - 1)
    last_n = N_NODES - last_off

    mesh = plsc.VectorSubcoreMesh(core_axis_name="c", subcore_axis_name="s")

    B = EDGE_BLOCK
    npairs = nblk // 2

    @functools.partial(
        pl.kernel,
        mesh=mesh,
        compiler_params=pltpu.CompilerParams(use_tc_tiling_on_sc=False),
        out_type=jax.ShapeDtypeStruct((SC_CORES, N_NODES, width), F32),
        scratch_types=[
            pltpu.VMEM((B,), jnp.int32), pltpu.VMEM((B,), jnp.int32),
            pltpu.VMEM((B,), jnp.int32), pltpu.VMEM((B,), jnp.int32),
            pltpu.VMEM((B,), jnp.int32), pltpu.VMEM((B,), jnp.int32),
            pltpu.VMEM((B, width), F32), pltpu.VMEM((B, width), F32),
            pltpu.VMEM((B, 16), F32), pltpu.VMEM((B, 16), F32),
            pltpu.VMEM((B, width), F32), pltpu.VMEM((B, width), F32),
            pltpu.VMEM_SHARED((N_NODES, width), F32),
            pltpu.SemaphoreType.DMA, pltpu.SemaphoreType.DMA,
            pltpu.SemaphoreType.DMA, pltpu.SemaphoreType.DMA,
            pltpu.SemaphoreType.DMA, pltpu.SemaphoreType.DMA,
        ],
    )
    def sc_edge_pass(t_hbm, a_hbm, src_hbm, dst_hbm, zero_hbm, p_hbm,
                     sidx0, sidx1, didx0, didx1, sdidx0, sdidx1,
                     rows0, rows1, arows0, arows1, wbuf0, wbuf1, acc,
                     semi0, semi1, semg0, semg1, sems0, sems1):
        cid = lax.axis_index("c")
        sid = lax.axis_index("s")
        wid = sid * SC_CORES + cid
        ebase = wid * per_w

        sidx = (sidx0, sidx1)
        didx = (didx0, didx1)
        sdidx = (sdidx0, sdidx1)
        rows = (rows0, rows1)
        arows = (arows0, arows1)
        wbuf = (wbuf0, wbuf1)
        semi = (semi0, semi1)
        semg = (semg0, semg1)
        sems = (sems0, sems1)

        def issue_idx(j, b):
            base = ebase + j * B
            pltpu.async_copy(src_hbm.at[pl.ds(base, B)], sidx[b], semi[b])
            pltpu.async_copy(dst_hbm.at[pl.ds(base, B)], didx[b], semi[b])

        def wait_idx(b):
            pltpu.make_async_copy(src_hbm.at[pl.ds(0, B)], sidx[b],
                                  semi[b]).wait()
            pltpu.make_async_copy(dst_hbm.at[pl.ds(0, B)], didx[b],
                                  semi[b]).wait()

        def issue_gather(b):
            pltpu.async_copy(t_hbm.at[sidx[b]], rows[b], semg[b])
            pltpu.async_copy(a_hbm.at[didx[b]], arows[b], semg[b])

        def wait_gather(b):
            pltpu.make_async_copy(t_hbm.at[sidx[b]], rows[b], semg[b]).wait()
            pltpu.make_async_copy(a_hbm.at[didx[b]], arows[b], semg[b]).wait()

        def copy_didx(b):
            for k in range(B // 16):
                sdidx[b][pl.ds(16 * k, 16)] = didx[b][pl.ds(16 * k, 16)]

        def compute(b):
            rb, ab, wb = rows[b], arows[b], wbuf[b]

            @plsc.parallel_loop(0, B, unroll=4)
            def _edge(i):
                z = rb[i, pl.ds(eoff, 16)] + ab[i, pl.ds(0, 16)]
                w = jnp.exp(jnp.maximum(z, 0.2 * z))
                wb[i, pl.ds(eoff, 16)] = w
                for k in range(nfeat):
                    wb[i, pl.ds(16 * k, 16)] = rb[i, pl.ds(16 * k, 16)] * w

        def issue_scatter(b):
            pltpu.async_copy(wbuf[b], acc.at[sdidx[b]], sems[b], add=True)

        def wait_scatter(b):
            pltpu.make_async_copy(wbuf[b], acc.at[sdidx[b]], sems[b]).wait()

        @pl.when(sid < SC_SUBCORES - 1)
        def _zero_main():
            pltpu.sync_copy(zero_hbm.at[pl.ds(sid * rps, rps)],
                            acc.at[pl.ds(sid * rps, rps)])

        @pl.when(sid == SC_SUBCORES - 1)
        def _zero_last():
            pltpu.sync_copy(zero_hbm.at[pl.ds(last_off, last_n)],
                            acc.at[pl.ds(last_off, last_n)])

        plsc.subcore_barrier()

        issue_idx(0, 0)
        issue_idx(1, 1)
        wait_idx(0)
        issue_gather(0)

        wait_gather(0)
        wait_idx(1)
        issue_gather(1)
        copy_didx(0)
        issue_idx(2, 0)
        compute(0)
        issue_scatter(0)

        wait_gather(1)
        wait_idx(0)
        issue_gather(0)
        copy_didx(1)
        issue_idx(3, 1)
        compute(1)
        issue_scatter(1)

        @pl.loop(1, npairs)
        def _pair(m):
            j0 = 2 * m
            wait_gather(0)
            wait_idx(1)
            issue_gather(1)
            wait_scatter(0)
            copy_didx(0)
            issue_idx(j0 + 2, 0)
            compute(0)
            issue_scatter(0)

            wait_gather(1)
            wait_idx(0)
            issue_gather(0)
            wait_scatter(1)
            copy_didx(1)

            @pl.when(m < npairs - 1)
            def _prefetch():
                issue_idx(j0 + 3, 1)

            compute(1)
            issue_scatter(1)

        wait_gather(0)
        wait_scatter(0)
        copy_didx(0)
        compute(0)
        issue_scatter(0)

        wait_scatter(1)
        wait_scatter(0)

        plsc.subcore_barrier()

        @pl.when(sid < SC_SUBCORES - 1)
        def _drain_main():
            pltpu.sync_copy(acc.at[pl.ds(sid * rps, rps)],
                            p_hbm.at[cid, pl.ds(sid * rps, rps)])

        @pl.when(sid == SC_SUBCORES - 1)
        def _drain_last():
            pltpu.sync_copy(acc.at[pl.ds(last_off, last_n)],
                            p_hbm.at[cid, pl.ds(last_off, last_n)])

    return sc_edge_pass


_sc_cache = {}


def _sc_pass(width, eoff, *args):
    key = (width, eoff)
    if key not in _sc_cache:
        _sc_cache[key] = _make_sc_edge_pass(width, eoff)
    return _sc_cache[key](*args)


def kernel(x, edge_index, W1, att_src1, att_dst1, b1,
           W2, att_src2, att_dst2, b2):
    perm = jnp.arange(HC)
    perm = (perm % 8) * 8 + perm // 8
    w1t = W1[:, perm]
    eye8 = jnp.eye(8, dtype=F32)
    ast = (att_src1[0].T[:, :, None] * eye8[None, :, :]).reshape(HC, HEADS)
    adt = (att_dst1[0].T[:, :, None] * eye8[None, :, :]).reshape(HC, HEADS)
    b1t = b1[perm].reshape(1, HC)
    w2t = W2[perm, :]
    att2m = jnp.concatenate(
        [att_src2[0, 0][:, None], att_dst2[0, 0][:, None]], axis=1)
    b2r = b2.reshape(1, NCLS)
    src = edge_index[0]
    dst = edge_index[1]
    zeros80 = jnp.zeros((N_NODES, 80), F32)
    zeros64 = jnp.zeros((N_NODES, 64), F32)

    t1, a1, init1 = _tc1(x, w1t, ast, adt)
    p = _sc_pass(80, 64, t1, a1, src, dst, zeros80)
    t2, a2, init2 = _tc2(p, init1, b1t, w2t, att2m)
    q = _sc_pass(64, 48, t2, a2, src, dst, zeros64)
    return _tc3(q, init2, b2r)


# speedup vs baseline: 136.2229x; 1.1872x over previous
import functools

import jax
import jax.numpy as jnp
from jax import lax
from jax.experimental import pallas as pl
from jax.experimental.pallas import tpu as pltpu
from jax.experimental.pallas import tpu_sc as plsc

F32 = jnp.float32

N_NODES = 10000
N_EDGES = 320000
F_IN = 128
HEADS = 8
CH1 = 8
HC = HEADS * CH1
NCLS = 40

SC_CORES = 2
SC_SUBCORES = 16
SC_WORKERS = SC_CORES * SC_SUBCORES
EDGE_BLOCK = 80

TC_BLOCK = 1000


def _tc1_body(x_ref, w_ref, ast_ref, adt_ref, s8_ref, e64_ref, e2_ref,
              edup_ref, t1_ref, a1_ref, init_ref):
    h = jnp.dot(x_ref[...], w_ref[...], preferred_element_type=F32)
    a_s = jnp.dot(h, ast_ref[...], preferred_element_type=F32)
    a_d = jnp.dot(h, adt_ref[...], preferred_element_type=F32)
    t1_ref[...] = (jnp.dot(h, e64_ref[...], preferred_element_type=F32)
                   + jnp.dot(a_s, e2_ref[...], preferred_element_type=F32))
    a1_ref[...] = jnp.dot(a_d, edup_ref[...], preferred_element_type=F32)
    z = a_s + a_d
    w = jnp.exp(jnp.maximum(z, 0.2 * z))
    w8 = jnp.dot(w, s8_ref[...], preferred_element_type=F32)
    init_ref[...] = (jnp.dot(h * w8, e64_ref[...], preferred_element_type=F32)
                     + jnp.dot(w, e2_ref[...], preferred_element_type=F32))


def _tc1(x, w1t, ast, adt, s8, e64, e2, edup):
    nb = N_NODES // TC_BLOCK
    return pl.pallas_call(
        _tc1_body,
        grid=(nb,),
        in_specs=[
            pl.BlockSpec((TC_BLOCK, F_IN), lambda i: (i, 0)),
            pl.BlockSpec((F_IN, HC), lambda i: (0, 0)),
            pl.BlockSpec((HC, HEADS), lambda i: (0, 0)),
            pl.BlockSpec((HC, HEADS), lambda i: (0, 0)),
            pl.BlockSpec((HEADS, HC), lambda i: (0, 0)),
            pl.BlockSpec((HC, 80), lambda i: (0, 0)),
            pl.BlockSpec((HEADS, 80), lambda i: (0, 0)),
            pl.BlockSpec((HEADS, 16), lambda i: (0, 0)),
        ],
        out_specs=[
            pl.BlockSpec((TC_BLOCK, 80), lambda i: (i, 0)),
            pl.BlockSpec((TC_BLOCK, 16), lambda i: (i, 0)),
            pl.BlockSpec((TC_BLOCK, 80), lambda i: (i, 0)),
        ],
        out_shape=[
            jax.ShapeDtypeStruct((N_NODES, 80), F32),
            jax.ShapeDtypeStruct((N_NODES, 16), F32),
            jax.ShapeDtypeStruct((N_NODES, 80), F32),
        ],
    )(x, w1t, ast, adt, s8, e64, e2, edup)


def _tc2_body(p_ref, init_ref, b1_ref, w2_ref, att2_ref, s8_ref, e40_ref,
              msrc_ref, t2_ref, a2_ref, init2_ref):
    nrow = p_ref.shape[1]
    acc = p_ref[0] + p_ref[1] + init_ref[...]
    rec = 1.0 / (acc[:, 64:72] + 1e-16)
    rec64 = jnp.dot(rec, s8_ref[...], preferred_element_type=F32)
    out1 = acc[:, :64] * rec64 + b1_ref[...]
    h2 = jnp.where(out1 > 0, out1, jnp.exp(out1) - 1.0)
    h2r = jnp.dot(h2, w2_ref[...], preferred_element_type=F32)
    a2 = jnp.dot(h2r, att2_ref[...], preferred_element_type=F32)
    a2s = a2[:, 0:1]
    a2d = a2[:, 1:2]
    t2_ref[...] = (jnp.dot(h2r, e40_ref[...], preferred_element_type=F32)
                   + jnp.broadcast_to(a2s, (nrow, 64)) * msrc_ref[...])
    a2_ref[...] = jnp.broadcast_to(a2d, (nrow, 16))
    z = a2s + a2d
    w = jnp.exp(jnp.maximum(z, 0.2 * z))
    init2_ref[...] = (jnp.dot(h2r * w, e40_ref[...], preferred_element_type=F32)
                      + jnp.broadcast_to(w, (nrow, 64)) * msrc_ref[...])


def _tc2(p, init1, b1t, w2t, att2m, s8, e40, msrc):
    nb = N_NODES // TC_BLOCK
    return pl.pallas_call(
        _tc2_body,
        grid=(nb,),
        in_specs=[
            pl.BlockSpec((SC_CORES, TC_BLOCK, 80), lambda i: (0, i, 0)),
            pl.BlockSpec((TC_BLOCK, 80), lambda i: (i, 0)),
            pl.BlockSpec((1, HC), lambda i: (0, 0)),
            pl.BlockSpec((HC, NCLS), lambda i: (0, 0)),
            pl.BlockSpec((NCLS, 2), lambda i: (0, 0)),
            pl.BlockSpec((HEADS, HC), lambda i: (0, 0)),
            pl.BlockSpec((NCLS, 64), lambda i: (0, 0)),
            pl.BlockSpec((1, 64), lambda i: (0, 0)),
        ],
        out_specs=[
            pl.BlockSpec((TC_BLOCK, 64), lambda i: (i, 0)),
            pl.BlockSpec((TC_BLOCK, 16), lambda i: (i, 0)),
            pl.BlockSpec((TC_BLOCK, 64), lambda i: (i, 0)),
        ],
        out_shape=[
            jax.ShapeDtypeStruct((N_NODES, 64), F32),
            jax.ShapeDtypeStruct((N_NODES, 16), F32),
            jax.ShapeDtypeStruct((N_NODES, 64), F32),
        ],
    )(p, init1, b1t, w2t, att2m, s8, e40, msrc)


def _tc3_body(q_ref, init2_ref, b2_ref, out_ref):
    acc = q_ref[0] + q_ref[1] + init2_ref[...]
    rec = 1.0 / (acc[:, 48:49] + 1e-16)
    logits = acc[:, :40] * jnp.broadcast_to(rec, (acc.shape[0], NCLS)) + b2_ref[...]
    m = jnp.max(logits, axis=1, keepdims=True)
    lse = jnp.log(jnp.sum(jnp.exp(logits - m), axis=1, keepdims=True)) + m
    out_ref[...] = logits - lse


def _tc3(q, init2, b2r):
    nb = N_NODES // TC_BLOCK
    return pl.pallas_call(
        _tc3_body,
        grid=(nb,),
        in_specs=[
            pl.BlockSpec((SC_CORES, TC_BLOCK, 64), lambda i: (0, i, 0)),
            pl.BlockSpec((TC_BLOCK, 64), lambda i: (i, 0)),
            pl.BlockSpec((1, NCLS), lambda i: (0, 0)),
        ],
        out_specs=pl.BlockSpec((TC_BLOCK, NCLS), lambda i: (i, 0)),
        out_shape=jax.ShapeDtypeStruct((N_NODES, NCLS), F32),
    )(q, init2, b2r)


def _make_sc_edge_pass(width, eoff):
    per_w = N_EDGES // SC_WORKERS
    nblk = per_w // EDGE_BLOCK
    nfeat = eoff // 16
    rps = 624
    last_off = rps * (SC_SUBCORES - 1)
    last_n = N_NODES - last_off

    mesh = plsc.VectorSubcoreMesh(core_axis_name="c", subcore_axis_name="s")

    B = EDGE_BLOCK
    npairs = nblk // 2

    @functools.partial(
        pl.kernel,
        mesh=mesh,
        compiler_params=pltpu.CompilerParams(use_tc_tiling_on_sc=False),
        out_type=jax.ShapeDtypeStruct((SC_CORES, N_NODES, width), F32),
        scratch_types=[
            pltpu.VMEM((B,), jnp.int32), pltpu.VMEM((B,), jnp.int32),
            pltpu.VMEM((B,), jnp.int32), pltpu.VMEM((B,), jnp.int32),
            pltpu.VMEM((B,), jnp.int32), pltpu.VMEM((B,), jnp.int32),
            pltpu.VMEM((B, width), F32), pltpu.VMEM((B, width), F32),
            pltpu.VMEM((B, 16), F32), pltpu.VMEM((B, 16), F32),
            pltpu.VMEM((B, width), F32), pltpu.VMEM((B, width), F32),
            pltpu.VMEM_SHARED((N_NODES, width), F32),
            pltpu.SemaphoreType.DMA, pltpu.SemaphoreType.DMA,
            pltpu.SemaphoreType.DMA, pltpu.SemaphoreType.DMA,
            pltpu.SemaphoreType.DMA, pltpu.SemaphoreType.DMA,
        ],
    )
    def sc_edge_pass(t_hbm, a_hbm, src_hbm, dst_hbm, zero_hbm, p_hbm,
                     sidx0, sidx1, didx0, didx1, sdidx0, sdidx1,
                     rows0, rows1, arows0, arows1, wbuf0, wbuf1, acc,
                     semi0, semi1, semg0, semg1, sems0, sems1):
        cid = lax.axis_index("c")
        sid = lax.axis_index("s")
        wid = sid * SC_CORES + cid
        ebase = wid * per_w

        sidx = (sidx0, sidx1)
        didx = (didx0, didx1)
        sdidx = (sdidx0, sdidx1)
        rows = (rows0, rows1)
        arows = (arows0, arows1)
        wbuf = (wbuf0, wbuf1)
        semi = (semi0, semi1)
        semg = (semg0, semg1)
        sems = (sems0, sems1)

        def issue_idx(j, b):
            base = ebase + j * B
            pltpu.async_copy(src_hbm.at[pl.ds(base, B)], sidx[b], semi[b])
            pltpu.async_copy(dst_hbm.at[pl.ds(base, B)], didx[b], semi[b])

        def wait_idx(b):
            pltpu.make_async_copy(src_hbm.at[pl.ds(0, B)], sidx[b],
                                  semi[b]).wait()
            pltpu.make_async_copy(dst_hbm.at[pl.ds(0, B)], didx[b],
                                  semi[b]).wait()

        def issue_gather(b):
            pltpu.async_copy(t_hbm.at[sidx[b]], rows[b], semg[b])
            pltpu.async_copy(a_hbm.at[didx[b]], arows[b], semg[b])

        def wait_gather(b):
            pltpu.make_async_copy(t_hbm.at[sidx[b]], rows[b], semg[b]).wait()
            pltpu.make_async_copy(a_hbm.at[didx[b]], arows[b], semg[b]).wait()

        def copy_didx(b):
            for k in range(B // 16):
                sdidx[b][pl.ds(16 * k, 16)] = didx[b][pl.ds(16 * k, 16)]

        def compute(b):
            rb, ab, wb = rows[b], arows[b], wbuf[b]

            @plsc.parallel_loop(0, B, unroll=8)
            def _edge(i):
                z = rb[i, pl.ds(eoff, 16)] + ab[i, pl.ds(0, 16)]
                w = jnp.exp(jnp.maximum(z, 0.2 * z))
                wb[i, pl.ds(eoff, 16)] = w
                for k in range(nfeat):
                    wb[i, pl.ds(16 * k, 16)] = rb[i, pl.ds(16 * k, 16)] * w

        def issue_scatter(b):
            pltpu.async_copy(wbuf[b], acc.at[sdidx[b]], sems[b], add=True)

        def wait_scatter(b):
            pltpu.make_async_copy(wbuf[b], acc.at[sdidx[b]], sems[b]).wait()

        @pl.when(sid < SC_SUBCORES - 1)
        def _zero_main():
            pltpu.sync_copy(zero_hbm.at[pl.ds(sid * rps, rps)],
                            acc.at[pl.ds(sid * rps, rps)])

        @pl.when(sid == SC_SUBCORES - 1)
        def _zero_last():
            pltpu.sync_copy(zero_hbm.at[pl.ds(last_off, last_n)],
                            acc.at[pl.ds(last_off, last_n)])

        plsc.subcore_barrier()

        issue_idx(0, 0)
        issue_idx(1, 1)
        wait_idx(0)
        issue_gather(0)

        wait_gather(0)
        wait_idx(1)
        issue_gather(1)
        copy_didx(0)
        issue_idx(2, 0)
        compute(0)
        issue_scatter(0)

        wait_gather(1)
        wait_idx(0)
        issue_gather(0)
        copy_didx(1)
        issue_idx(3, 1)
        compute(1)
        issue_scatter(1)

        @pl.loop(1, npairs)
        def _pair(m):
            j0 = 2 * m
            wait_gather(0)
            wait_idx(1)
            issue_gather(1)
            wait_scatter(0)
            copy_didx(0)
            issue_idx(j0 + 2, 0)
            compute(0)
            issue_scatter(0)

            wait_gather(1)
            wait_idx(0)
            issue_gather(0)
            wait_scatter(1)
            copy_didx(1)

            @pl.when(m < npairs - 1)
            def _prefetch():
                issue_idx(j0 + 3, 1)

            compute(1)
            issue_scatter(1)

        wait_gather(0)
        wait_scatter(0)
        copy_didx(0)
        compute(0)
        issue_scatter(0)

        wait_scatter(1)
        wait_scatter(0)

        plsc.subcore_barrier()

        @pl.when(sid < SC_SUBCORES - 1)
        def _drain_main():
            pltpu.sync_copy(acc.at[pl.ds(sid * rps, rps)],
                            p_hbm.at[cid, pl.ds(sid * rps, rps)])

        @pl.when(sid == SC_SUBCORES - 1)
        def _drain_last():
            pltpu.sync_copy(acc.at[pl.ds(last_off, last_n)],
                            p_hbm.at[cid, pl.ds(last_off, last_n)])

    return sc_edge_pass


_sc_cache = {}


def _sc_pass(width, eoff, *args):
    key = (width, eoff)
    if key not in _sc_cache:
        _sc_cache[key] = _make_sc_edge_pass(width, eoff)
    return _sc_cache[key](*args)


def kernel(x, edge_index, W1, att_src1, att_dst1, b1,
           W2, att_src2, att_dst2, b2):
    perm = jnp.arange(HC)
    perm = (perm % 8) * 8 + perm // 8
    w1t = W1[:, perm]
    eye8 = jnp.eye(8, dtype=F32)
    ast = (att_src1[0].T[:, :, None] * eye8[None, :, :]).reshape(HC, HEADS)
    adt = (att_dst1[0].T[:, :, None] * eye8[None, :, :]).reshape(HC, HEADS)
    b1t = b1[perm].reshape(1, HC)
    w2t = W2[perm, :]
    att2m = jnp.concatenate(
        [att_src2[0, 0][:, None], att_dst2[0, 0][:, None]], axis=1)
    b2r = b2.reshape(1, NCLS)
    src = edge_index[0]
    dst = edge_index[1]
    zeros80 = jnp.zeros((N_NODES, 80), F32)
    zeros64 = jnp.zeros((N_NODES, 64), F32)

    col = jnp.arange(80)
    s8 = (jnp.arange(8)[:, None] == (jnp.arange(64)[None, :] % 8)).astype(F32)
    e64 = (jnp.arange(64)[:, None] == col[None, :]).astype(F32)
    e2 = ((jnp.arange(8)[:, None] + 64 == col[None, :])
          | (jnp.arange(8)[:, None] + 72 == col[None, :])).astype(F32)
    edup = ((jnp.arange(8)[:, None] == jnp.arange(16)[None, :])
            | (jnp.arange(8)[:, None] + 8 == jnp.arange(16)[None, :])).astype(F32)
    e40 = (jnp.arange(NCLS)[:, None] == jnp.arange(64)[None, :]).astype(F32)
    msrc = (jnp.arange(64)[None, :] >= 48).astype(F32)

    t1, a1, init1 = _tc1(x, w1t, ast, adt, s8, e64, e2, edup)
    p = _sc_pass(80, 64, t1, a1, src, dst, zeros80)
    t2, a2, init2 = _tc2(p, init1, b1t, w2t, att2m, s8, e40, msrc)
    q = _sc_pass(64, 48, t2, a2, src, dst, zeros64)
    return _tc3(q, init2, b2r)


# speedup vs baseline: 147.0343x; 1.0794x over previous
import functools

import jax
import jax.numpy as jnp
from jax import lax
from jax.experimental import pallas as pl
from jax.experimental.pallas import tpu as pltpu
from jax.experimental.pallas import tpu_sc as plsc

F32 = jnp.float32

N_NODES = 10000
N_EDGES = 320000
F_IN = 128
HEADS = 8
CH1 = 8
HC = HEADS * CH1
NCLS = 40

SC_CORES = 2
SC_SUBCORES = 16
SC_WORKERS = SC_CORES * SC_SUBCORES
EDGE_BLOCK = 80

TC_BLOCK = 1000


def _tc1_body(x_ref, w_ref, ast_ref, adt_ref, s8_ref, e64_ref, e2_ref,
              edup_ref, t1_ref, a1_ref, init_ref):
    h = jnp.dot(x_ref[...], w_ref[...], preferred_element_type=F32)
    a_s = jnp.dot(h, ast_ref[...], preferred_element_type=F32)
    a_d = jnp.dot(h, adt_ref[...], preferred_element_type=F32)
    t1_ref[...] = (jnp.dot(h, e64_ref[...], preferred_element_type=F32)
                   + jnp.dot(a_s, e2_ref[...], preferred_element_type=F32))
    a1_ref[...] = jnp.dot(a_d, edup_ref[...], preferred_element_type=F32)
    z = a_s + a_d
    w = jnp.exp(jnp.maximum(z, 0.2 * z))
    w8 = jnp.dot(w, s8_ref[...], preferred_element_type=F32)
    init_ref[...] = (jnp.dot(h * w8, e64_ref[...], preferred_element_type=F32)
                     + jnp.dot(w, e2_ref[...], preferred_element_type=F32))


def _tc1(x, w1t, ast, adt, s8, e64, e2, edup):
    nb = N_NODES // TC_BLOCK
    return pl.pallas_call(
        _tc1_body,
        grid=(nb,),
        in_specs=[
            pl.BlockSpec((TC_BLOCK, F_IN), lambda i: (i, 0)),
            pl.BlockSpec((F_IN, HC), lambda i: (0, 0)),
            pl.BlockSpec((HC, HEADS), lambda i: (0, 0)),
            pl.BlockSpec((HC, HEADS), lambda i: (0, 0)),
            pl.BlockSpec((HEADS, HC), lambda i: (0, 0)),
            pl.BlockSpec((HC, 80), lambda i: (0, 0)),
            pl.BlockSpec((HEADS, 80), lambda i: (0, 0)),
            pl.BlockSpec((HEADS, 16), lambda i: (0, 0)),
        ],
        out_specs=[
            pl.BlockSpec((TC_BLOCK, 80), lambda i: (i, 0)),
            pl.BlockSpec((TC_BLOCK, 16), lambda i: (i, 0)),
            pl.BlockSpec((TC_BLOCK, 80), lambda i: (i, 0)),
        ],
        out_shape=[
            jax.ShapeDtypeStruct((N_NODES, 80), F32),
            jax.ShapeDtypeStruct((N_NODES, 16), F32),
            jax.ShapeDtypeStruct((N_NODES, 80), F32),
        ],
    )(x, w1t, ast, adt, s8, e64, e2, edup)


def _tc2_body(p_ref, init_ref, b1_ref, w2_ref, att2_ref, s8_ref, e40_ref,
              msrc_ref, t2_ref, a2_ref, init2_ref):
    nrow = p_ref.shape[1]
    acc = p_ref[0, :, :80] + p_ref[1, :, :80] + init_ref[...]
    rec = 1.0 / (acc[:, 64:72] + 1e-16)
    rec64 = jnp.dot(rec, s8_ref[...], preferred_element_type=F32)
    out1 = acc[:, :64] * rec64 + b1_ref[...]
    h2 = jnp.where(out1 > 0, out1, jnp.exp(out1) - 1.0)
    h2r = jnp.dot(h2, w2_ref[...], preferred_element_type=F32)
    a2 = jnp.dot(h2r, att2_ref[...], preferred_element_type=F32)
    a2s = a2[:, 0:1]
    a2d = a2[:, 1:2]
    t2_ref[...] = (jnp.dot(h2r, e40_ref[...], preferred_element_type=F32)
                   + jnp.broadcast_to(a2s, (nrow, 64)) * msrc_ref[...])
    a2_ref[...] = jnp.broadcast_to(a2d, (nrow, 16))
    z = a2s + a2d
    w = jnp.exp(jnp.maximum(z, 0.2 * z))
    init2_ref[...] = (jnp.dot(h2r * w, e40_ref[...], preferred_element_type=F32)
                      + jnp.broadcast_to(w, (nrow, 64)) * msrc_ref[...])


def _tc2(p, init1, b1t, w2t, att2m, s8, e40, msrc):
    nb = N_NODES // TC_BLOCK
    return pl.pallas_call(
        _tc2_body,
        grid=(nb,),
        in_specs=[
            pl.BlockSpec((SC_CORES, TC_BLOCK, 128), lambda i: (0, i, 0)),
            pl.BlockSpec((TC_BLOCK, 80), lambda i: (i, 0)),
            pl.BlockSpec((1, HC), lambda i: (0, 0)),
            pl.BlockSpec((HC, NCLS), lambda i: (0, 0)),
            pl.BlockSpec((NCLS, 2), lambda i: (0, 0)),
            pl.BlockSpec((HEADS, HC), lambda i: (0, 0)),
            pl.BlockSpec((NCLS, 64), lambda i: (0, 0)),
            pl.BlockSpec((1, 64), lambda i: (0, 0)),
        ],
        out_specs=[
            pl.BlockSpec((TC_BLOCK, 64), lambda i: (i, 0)),
            pl.BlockSpec((TC_BLOCK, 16), lambda i: (i, 0)),
            pl.BlockSpec((TC_BLOCK, 64), lambda i: (i, 0)),
        ],
        out_shape=[
            jax.ShapeDtypeStruct((N_NODES, 64), F32),
            jax.ShapeDtypeStruct((N_NODES, 16), F32),
            jax.ShapeDtypeStruct((N_NODES, 64), F32),
        ],
    )(p, init1, b1t, w2t, att2m, s8, e40, msrc)


def _tc3_body(q_ref, init2_ref, b2_ref, out_ref):
    acc = q_ref[0, :, :64] + q_ref[1, :, :64] + init2_ref[...]
    rec = 1.0 / (acc[:, 48:49] + 1e-16)
    logits = acc[:, :40] * jnp.broadcast_to(rec, (acc.shape[0], NCLS)) + b2_ref[...]
    m = jnp.max(logits, axis=1, keepdims=True)
    lse = jnp.log(jnp.sum(jnp.exp(logits - m), axis=1, keepdims=True)) + m
    out_ref[...] = logits - lse


def _tc3(q, init2, b2r):
    nb = N_NODES // TC_BLOCK
    return pl.pallas_call(
        _tc3_body,
        grid=(nb,),
        in_specs=[
            pl.BlockSpec((SC_CORES, TC_BLOCK, 128), lambda i: (0, i, 0)),
            pl.BlockSpec((TC_BLOCK, 64), lambda i: (i, 0)),
            pl.BlockSpec((1, NCLS), lambda i: (0, 0)),
        ],
        out_specs=pl.BlockSpec((TC_BLOCK, NCLS), lambda i: (i, 0)),
        out_shape=jax.ShapeDtypeStruct((N_NODES, NCLS), F32),
    )(q, init2, b2r)


def _make_sc_edge_pass(width, eoff):
    per_w = N_EDGES // SC_WORKERS
    nblk = per_w // EDGE_BLOCK
    nfeat = eoff // 16
    rps = 624
    last_off = rps * (SC_SUBCORES - 1)
    last_n = N_NODES - last_off

    mesh = plsc.VectorSubcoreMesh(core_axis_name="c", subcore_axis_name="s")

    B = EDGE_BLOCK
    npairs = nblk // 2

    @functools.partial(
        pl.kernel,
        mesh=mesh,
        compiler_params=pltpu.CompilerParams(use_tc_tiling_on_sc=False),
        out_type=jax.ShapeDtypeStruct((SC_CORES, N_NODES, 128), F32),
        scratch_types=[
            pltpu.VMEM((B,), jnp.int32), pltpu.VMEM((B,), jnp.int32),
            pltpu.VMEM((B,), jnp.int32), pltpu.VMEM((B,), jnp.int32),
            pltpu.VMEM((B,), jnp.int32), pltpu.VMEM((B,), jnp.int32),
            pltpu.VMEM((B, width), F32), pltpu.VMEM((B, width), F32),
            pltpu.VMEM((B, 16), F32), pltpu.VMEM((B, 16), F32),
            pltpu.VMEM((B, width), F32), pltpu.VMEM((B, width), F32),
            pltpu.VMEM_SHARED((N_NODES, width), F32),
            pltpu.SemaphoreType.DMA, pltpu.SemaphoreType.DMA,
            pltpu.SemaphoreType.DMA, pltpu.SemaphoreType.DMA,
            pltpu.SemaphoreType.DMA, pltpu.SemaphoreType.DMA,
        ],
    )
    def sc_edge_pass(t_hbm, a_hbm, ei_hbm, zero_hbm, p_hbm,
                     sidx0, sidx1, didx0, didx1, sdidx0, sdidx1,
                     rows0, rows1, arows0, arows1, wbuf0, wbuf1, acc,
                     semi0, semi1, semg0, semg1, sems0, sems1):
        cid = lax.axis_index("c")
        sid = lax.axis_index("s")
        wid = sid * SC_CORES + cid
        ebase = wid * per_w

        sidx = (sidx0, sidx1)
        didx = (didx0, didx1)
        sdidx = (sdidx0, sdidx1)
        rows = (rows0, rows1)
        arows = (arows0, arows1)
        wbuf = (wbuf0, wbuf1)
        semi = (semi0, semi1)
        semg = (semg0, semg1)
        sems = (sems0, sems1)

        def issue_idx(j, b):
            base = ebase + j * B
            pltpu.async_copy(ei_hbm.at[0, pl.ds(base, B)], sidx[b], semi[b])
            pltpu.async_copy(ei_hbm.at[1, pl.ds(base, B)], didx[b], semi[b])

        def wait_idx(b):
            pltpu.make_async_copy(ei_hbm.at[0, pl.ds(0, B)], sidx[b],
                                  semi[b]).wait()
            pltpu.make_async_copy(ei_hbm.at[1, pl.ds(0, B)], didx[b],
                                  semi[b]).wait()

        def issue_gather(b):
            pltpu.async_copy(t_hbm.at[sidx[b]], rows[b], semg[b])
            pltpu.async_copy(a_hbm.at[didx[b]], arows[b], semg[b])

        def wait_gather(b):
            pltpu.make_async_copy(t_hbm.at[sidx[b]], rows[b], semg[b]).wait()
            pltpu.make_async_copy(a_hbm.at[didx[b]], arows[b], semg[b]).wait()

        def copy_didx(b):
            for k in range(B // 16):
                sdidx[b][pl.ds(16 * k, 16)] = didx[b][pl.ds(16 * k, 16)]

        def compute(b):
            rb, ab, wb = rows[b], arows[b], wbuf[b]

            @plsc.parallel_loop(0, B, unroll=8)
            def _edge(i):
                z = rb[i, pl.ds(eoff, 16)] + ab[i, pl.ds(0, 16)]
                w = jnp.exp(jnp.maximum(z, 0.2 * z))
                wb[i, pl.ds(eoff, 16)] = w
                for k in range(nfeat):
                    wb[i, pl.ds(16 * k, 16)] = rb[i, pl.ds(16 * k, 16)] * w

        def issue_scatter(b):
            pltpu.async_copy(wbuf[b], acc.at[sdidx[b]], sems[b], add=True)

        def wait_scatter(b):
            pltpu.make_async_copy(wbuf[b], acc.at[sdidx[b]], sems[b]).wait()

        @pl.when(sid < SC_SUBCORES - 1)
        def _zero_main():
            pltpu.sync_copy(zero_hbm.at[pl.ds(sid * rps, rps), pl.ds(0, width)],
                            acc.at[pl.ds(sid * rps, rps)])

        @pl.when(sid == SC_SUBCORES - 1)
        def _zero_last():
            pltpu.sync_copy(zero_hbm.at[pl.ds(last_off, last_n), pl.ds(0, width)],
                            acc.at[pl.ds(last_off, last_n)])

        plsc.subcore_barrier()

        issue_idx(0, 0)
        issue_idx(1, 1)
        wait_idx(0)
        issue_gather(0)

        wait_gather(0)
        wait_idx(1)
        issue_gather(1)
        copy_didx(0)
        issue_idx(2, 0)
        compute(0)
        issue_scatter(0)

        wait_gather(1)
        wait_idx(0)
        issue_gather(0)
        copy_didx(1)
        issue_idx(3, 1)
        compute(1)
        issue_scatter(1)

        @pl.loop(1, npairs)
        def _pair(m):
            j0 = 2 * m
            wait_gather(0)
            wait_idx(1)
            issue_gather(1)
            wait_scatter(0)
            copy_didx(0)
            issue_idx(j0 + 2, 0)
            compute(0)
            issue_scatter(0)

            wait_gather(1)
            wait_idx(0)
            issue_gather(0)
            wait_scatter(1)
            copy_didx(1)

            @pl.when(m < npairs - 1)
            def _prefetch():
                issue_idx(j0 + 3, 1)

            compute(1)
            issue_scatter(1)

        wait_gather(0)
        wait_scatter(0)
        copy_didx(0)
        compute(0)
        issue_scatter(0)

        wait_scatter(1)
        wait_scatter(0)

        plsc.subcore_barrier()

        @pl.when(sid < SC_SUBCORES - 1)
        def _drain_main():
            pltpu.sync_copy(acc.at[pl.ds(sid * rps, rps)],
                            p_hbm.at[cid, pl.ds(sid * rps, rps),
                                     pl.ds(0, width)])

        @pl.when(sid == SC_SUBCORES - 1)
        def _drain_last():
            pltpu.sync_copy(acc.at[pl.ds(last_off, last_n)],
                            p_hbm.at[cid, pl.ds(last_off, last_n),
                                     pl.ds(0, width)])

    return sc_edge_pass


_sc_cache = {}


def _sc_pass(width, eoff, *args):
    key = (width, eoff)
    if key not in _sc_cache:
        _sc_cache[key] = _make_sc_edge_pass(width, eoff)
    return _sc_cache[key](*args)


def kernel(x, edge_index, W1, att_src1, att_dst1, b1,
           W2, att_src2, att_dst2, b2):
    perm = jnp.arange(HC)
    perm = (perm % 8) * 8 + perm // 8
    w1t = W1[:, perm]
    eye8 = jnp.eye(8, dtype=F32)
    ast = (att_src1[0].T[:, :, None] * eye8[None, :, :]).reshape(HC, HEADS)
    adt = (att_dst1[0].T[:, :, None] * eye8[None, :, :]).reshape(HC, HEADS)
    b1t = b1[perm].reshape(1, HC)
    w2t = W2[perm, :]
    att2m = jnp.concatenate(
        [att_src2[0, 0][:, None], att_dst2[0, 0][:, None]], axis=1)
    b2r = b2.reshape(1, NCLS)
    zeros80 = jnp.zeros((N_NODES, 80), F32)

    col = jnp.arange(80)
    s8 = (jnp.arange(8)[:, None] == (jnp.arange(64)[None, :] % 8)).astype(F32)
    e64 = (jnp.arange(64)[:, None] == col[None, :]).astype(F32)
    e2 = ((jnp.arange(8)[:, None] + 64 == col[None, :])
          | (jnp.arange(8)[:, None] + 72 == col[None, :])).astype(F32)
    edup = ((jnp.arange(8)[:, None] == jnp.arange(16)[None, :])
            | (jnp.arange(8)[:, None] + 8 == jnp.arange(16)[None, :])).astype(F32)
    e40 = (jnp.arange(NCLS)[:, None] == jnp.arange(64)[None, :]).astype(F32)
    msrc = (jnp.arange(64)[None, :] >= 48).astype(F32)

    t1, a1, init1 = _tc1(x, w1t, ast, adt, s8, e64, e2, edup)
    p = _sc_pass(80, 64, t1, a1, edge_index, zeros80)
    t2, a2, init2 = _tc2(p, init1, b1t, w2t, att2m, s8, e40, msrc)
    q = _sc_pass(64, 48, t2, a2, edge_index, zeros80)
    return _tc3(q, init2, b2r)
